# baseline (speedup 1.0000x reference)
import jax
import jax.numpy as jnp
from jax import lax
from jax.experimental import pallas as pl
from jax.experimental.pallas import tpu as pltpu
from jax.experimental.pallas import tpu_sc as plsc

_MIN_PIXELS = 128
_RATIO = 16
_H = 32
_HW = _H * _H
_L = 16
_CHUNK = 128


def _pool_body(mask_ref, sel_ref):
    bs = mask_ref.shape[0]
    r = lax.broadcasted_iota(jnp.int32, (_H, 512), 0)
    c = lax.broadcasted_iota(jnp.int32, (_H, 512), 1)
    pool = (c // _RATIO == r).astype(jnp.float32)
    for b in range(bs):
        m = mask_ref[b].astype(jnp.float32)
        t = jnp.dot(pool, m, preferred_element_type=jnp.float32)
        cnt = lax.dot_general(t, pool, (((1,), (1,)), ((), ())),
                              preferred_element_type=jnp.float32)
        sel_ref[b] = (cnt >= _MIN_PIXELS).astype(jnp.float32)


def _transpose_body(emb_ref, out_ref):
    out_ref[0] = emb_ref[0].T


def _sc_mean_body(ids_hbm, sel_hbm, embt_hbm, out_hbm,
                  ids_v, selchunk_v, selrow_v, rowbuf_v, acc_v,
                  outrow_v, seglist_s):
    img = lax.axis_index("s")
    parity = lax.axis_index("c")
    S = ids_v.shape[0]
    C = acc_v.shape[1]
    nvec = C // _L

    pltpu.sync_copy(ids_hbm, ids_v)

    def scan_body(g, cnt):
        idv = ids_v[pl.ds(g * _L, _L)]
        for l in range(_L):
            s = g * _L + l
            match = jnp.logical_and(idv[l] == img,
                                    lax.rem(s, 2) == parity)

            @pl.when(match)
            def _(cnt=cnt, s=s):
                seglist_s[cnt] = s

            cnt = cnt + match.astype(jnp.int32)
        return cnt

    nseg = lax.fori_loop(0, S // _L, scan_body, jnp.int32(0))

    def zero_body(j, _):
        s = seglist_s[j]
        for v in range(nvec):
            acc_v[s, pl.ds(v * _L, _L)] = jnp.zeros((_L,), jnp.float32)
        return 0

    lax.fori_loop(0, nseg, zero_body, 0)

    def chunk_body(chunk, _):
        pltpu.sync_copy(embt_hbm.at[img, pl.ds(chunk * _CHUNK, _CHUNK), :],
                        rowbuf_v)

        def seg_body(j, _):
            s = seglist_s[j]
            pltpu.sync_copy(sel_hbm.at[s, pl.ds(chunk * _CHUNK, _CHUNK)],
                            selchunk_v)

            def g_body(g, accs):
                pv = selchunk_v[pl.ds(g * _L, _L)]
                for l in range(_L):
                    wgt = pv[l]
                    p = g * _L + l
                    accs = tuple(
                        a + rowbuf_v[p, pl.ds(v * _L, _L)] * wgt
                        for v, a in enumerate(accs))
                return accs

            accs = tuple(acc_v[s, pl.ds(v * _L, _L)] for v in range(nvec))
            accs = lax.fori_loop(0, _CHUNK // _L, g_body, accs)
            for v in range(nvec):
                acc_v[s, pl.ds(v * _L, _L)] = accs[v]
            return 0

        lax.fori_loop(0, nseg, seg_body, 0)
        return 0

    lax.fori_loop(0, _HW // _CHUNK, chunk_body, 0)

    def fin_body(j, _):
        s = seglist_s[j]
        pltpu.sync_copy(acc_v.at[s], out_hbm.at[s])
        return 0

    lax.fori_loop(0, nseg, fin_body, 0)


def _div_body(sel_ref, num_ref, out_ref):
    den = jnp.sum(sel_ref[...], axis=1, keepdims=True)
    out_ref[...] = num_ref[...] / den


def kernel(binary_masks, image_ids, relative_segment_ids, coords,
           sam_encoder_embeddings):
    S = binary_masks.shape[0]
    n_envs = sam_encoder_embeddings.shape[0]
    C = sam_encoder_embeddings.shape[2]
    masks = binary_masks.reshape(S, 512, 512)
    emb = sam_encoder_embeddings.reshape(n_envs, C, _HW)

    BS = 4
    sel = pl.pallas_call(
        _pool_body,
        grid=(S // BS,),
        in_specs=[pl.BlockSpec((BS, 512, 512), lambda i: (i, 0, 0))],
        out_specs=pl.BlockSpec((BS, _H, _H), lambda i: (i, 0, 0)),
        out_shape=jax.ShapeDtypeStruct((S, _H, _H), jnp.float32),
    )(masks)
    sel2 = sel.reshape(S, _HW)

    embt = pl.pallas_call(
        _transpose_body,
        grid=(n_envs,),
        in_specs=[pl.BlockSpec((1, C, _HW), lambda i: (i, 0, 0))],
        out_specs=pl.BlockSpec((1, _HW, C), lambda i: (i, 0, 0)),
        out_shape=jax.ShapeDtypeStruct((n_envs, _HW, C), jnp.float32),
    )(emb)

    mesh = plsc.VectorSubcoreMesh(core_axis_name="c", subcore_axis_name="s")
    num = pl.kernel(
        _sc_mean_body,
        out_type=jax.ShapeDtypeStruct((S, C), jnp.float32),
        mesh=mesh,
        scratch_types=[
            pltpu.VMEM((S,), jnp.int32),
            pltpu.VMEM((_CHUNK,), jnp.float32),
            pltpu.VMEM((_HW,), jnp.float32),
            pltpu.VMEM((_CHUNK, C), jnp.float32),
            pltpu.VMEM((S, C), jnp.float32),
            pltpu.VMEM((C,), jnp.float32),
            pltpu.SMEM((S,), jnp.int32),
        ],
    )(image_ids, sel2, embt)

    segs = pl.pallas_call(
        _div_body,
        in_specs=[
            pl.BlockSpec((S, _HW), lambda: (0, 0)),
            pl.BlockSpec((S, C), lambda: (0, 0)),
        ],
        out_specs=pl.BlockSpec((S, C), lambda: (0, 0)),
        out_shape=jax.ShapeDtypeStruct((S, C), jnp.float32),
    )(sel2, num)

    is_latent_tokens = jnp.zeros((S,), dtype=bool)
    return (image_ids, relative_segment_ids, is_latent_tokens, segs, coords)


# speedup vs baseline: 1.2540x; 1.2540x over previous
import jax
import jax.numpy as jnp
from jax import lax
from jax.experimental import pallas as pl
from jax.experimental.pallas import tpu as pltpu
from jax.experimental.pallas import tpu_sc as plsc

_MIN_PIXELS = 128
_RATIO = 16
_H = 32
_HW = _H * _H
_L = 16
_CHUNK = 128


def _pool_body(mask_ref, sel_ref):
    bs = mask_ref.shape[0]
    r = lax.broadcasted_iota(jnp.int32, (_H, 512), 0)
    c = lax.broadcasted_iota(jnp.int32, (_H, 512), 1)
    pool = (c // _RATIO == r).astype(jnp.float32)
    for b in range(bs):
        m = mask_ref[b].astype(jnp.float32)
        t = jnp.dot(pool, m, preferred_element_type=jnp.float32)
        cnt = lax.dot_general(t, pool, (((1,), (1,)), ((), ())),
                              preferred_element_type=jnp.float32)
        sel_ref[b] = (cnt >= _MIN_PIXELS).astype(jnp.float32)


def _transpose_body(emb_ref, out_ref):
    out_ref[0] = emb_ref[0].T


_NBUF = 2
_BATCH = 16


def _sc_mean_body(ids_hbm, sel_hbm, embt_hbm, out_hbm,
                  ids_v, selbatch_v, rowbuf_v, acc_v, seglist_s, sems):
    img = lax.axis_index("s")
    parity = lax.axis_index("c")
    S = ids_v.shape[0]
    C = acc_v.shape[1]
    nvec = C // _L
    nchunk = _HW // _CHUNK

    pltpu.sync_copy(ids_hbm, ids_v)

    def scan_body(g, cnt):
        idv = ids_v[pl.ds(g * _L, _L)]
        for l in range(_L):
            s = g * _L + l
            match = jnp.logical_and(idv[l] == img,
                                    lax.rem(s, 2) == parity)

            @pl.when(match)
            def _(cnt=cnt, s=s):
                seglist_s[cnt] = s

            cnt = cnt + match.astype(jnp.int32)
        return cnt

    nseg = lax.fori_loop(0, S // _L, scan_body, jnp.int32(0))
    nbatch = (nseg + _BATCH - 1) // _BATCH

    def emb_copy(chunk, buf):
        return pltpu.make_async_copy(
            embt_hbm.at[img, pl.ds(chunk * _CHUNK, _CHUNK), :],
            rowbuf_v.at[buf], sems.at[buf])

    def batch_body(b, _):
        bsize = jnp.minimum(nseg - b * _BATCH, _BATCH)

        def ld_body(j, _):
            s = seglist_s[b * _BATCH + j]
            pltpu.sync_copy(sel_hbm.at[s], selbatch_v.at[j])
            for v in range(nvec):
                acc_v[j, pl.ds(v * _L, _L)] = jnp.zeros((_L,), jnp.float32)
            return 0

        lax.fori_loop(0, bsize, ld_body, 0)

        emb_copy(0, 0).start()

        def chunk_body(chunk, _):
            buf = lax.rem(chunk, _NBUF)
            emb_copy(chunk, buf).wait()

            @pl.when(chunk + 1 < nchunk)
            def _():
                emb_copy(chunk + 1, lax.rem(chunk + 1, _NBUF)).start()

            def seg_body(j, _):
                def g_body(g, accs):
                    pv = selbatch_v[j, pl.ds(chunk * _CHUNK + g * _L, _L)]
                    for l in range(_L):
                        wgt = pv[l]
                        accs = tuple(
                            a + rowbuf_v[buf, g * _L + l, pl.ds(v * _L, _L)] * wgt
                            for v, a in enumerate(accs))
                    return accs

                accs = tuple(acc_v[j, pl.ds(v * _L, _L)] for v in range(nvec))
                accs = lax.fori_loop(0, _CHUNK // _L, g_body, accs)
                for v in range(nvec):
                    acc_v[j, pl.ds(v * _L, _L)] = accs[v]
                return 0

            lax.fori_loop(0, bsize, seg_body, 0)
            return 0

        lax.fori_loop(0, nchunk, chunk_body, 0)

        def fin_body(j, _):
            s = seglist_s[b * _BATCH + j]
            pltpu.sync_copy(acc_v.at[j], out_hbm.at[s])
            return 0

        lax.fori_loop(0, bsize, fin_body, 0)
        return 0

    lax.fori_loop(0, nbatch, batch_body, 0)


def _div_body(sel_ref, num_ref, out_ref):
    den = jnp.sum(sel_ref[...], axis=1, keepdims=True)
    out_ref[...] = num_ref[...] / den


def kernel(binary_masks, image_ids, relative_segment_ids, coords,
           sam_encoder_embeddings):
    S = binary_masks.shape[0]
    n_envs = sam_encoder_embeddings.shape[0]
    C = sam_encoder_embeddings.shape[2]
    masks = binary_masks.reshape(S, 512, 512)
    emb = sam_encoder_embeddings.reshape(n_envs, C, _HW)

    BS = 4
    sel = pl.pallas_call(
        _pool_body,
        grid=(S // BS,),
        in_specs=[pl.BlockSpec((BS, 512, 512), lambda i: (i, 0, 0))],
        out_specs=pl.BlockSpec((BS, _H, _H), lambda i: (i, 0, 0)),
        out_shape=jax.ShapeDtypeStruct((S, _H, _H), jnp.float32),
    )(masks)
    sel2 = sel.reshape(S, _HW)

    embt = pl.pallas_call(
        _transpose_body,
        grid=(n_envs,),
        in_specs=[pl.BlockSpec((1, C, _HW), lambda i: (i, 0, 0))],
        out_specs=pl.BlockSpec((1, _HW, C), lambda i: (i, 0, 0)),
        out_shape=jax.ShapeDtypeStruct((n_envs, _HW, C), jnp.float32),
    )(emb)

    mesh = plsc.VectorSubcoreMesh(core_axis_name="c", subcore_axis_name="s")
    num = pl.kernel(
        _sc_mean_body,
        out_type=jax.ShapeDtypeStruct((S, C), jnp.float32),
        mesh=mesh,
        scratch_types=[
            pltpu.VMEM((S,), jnp.int32),
            pltpu.VMEM((_BATCH, _HW), jnp.float32),
            pltpu.VMEM((_NBUF, _CHUNK, C), jnp.float32),
            pltpu.VMEM((_BATCH, C), jnp.float32),
            pltpu.SMEM((S,), jnp.int32),
            pltpu.SemaphoreType.DMA((_NBUF,)),
        ],
    )(image_ids, sel2, embt)

    segs = pl.pallas_call(
        _div_body,
        in_specs=[
            pl.BlockSpec((S, _HW), lambda: (0, 0)),
            pl.BlockSpec((S, C), lambda: (0, 0)),
        ],
        out_specs=pl.BlockSpec((S, C), lambda: (0, 0)),
        out_shape=jax.ShapeDtypeStruct((S, C), jnp.float32),
    )(sel2, num)

    is_latent_tokens = jnp.zeros((S,), dtype=bool)
    return (image_ids, relative_segment_ids, is_latent_tokens, segs, coords)
